# baseline (speedup 1.0000x reference)
import functools

import jax
import jax.numpy as jnp
from jax import lax
from jax.experimental import pallas as pl
from jax.experimental.pallas import tpu as pltpu
from jax.experimental.pallas import tpu_sc as plsc

NW = 32
NB = 2
SB_TOTAL = 200 * 16
SB_PER_W = SB_TOTAL // NW


def _emb_body(x5_hbm, w2_hbm, out5_hbm,
              idxv, hbuf, obuf, widebuf, outbuf,
              gsem0, gsem1, ssem0, ssem1):
    wid = lax.axis_index("s") * 2 + lax.axis_index("c")
    iota16 = lax.iota(jnp.int32, 16)
    gsems = (gsem0, gsem1)
    ssems = (ssem0, ssem1)

    def coords(t):
        s = wid * SB_PER_W + t
        j = s // 16
        i2 = s % 16
        return j // 8, j % 8, j, i2

    def stage_a(t, slot):
        jr, jj, j, i2 = coords(t)
        pltpu.sync_copy(x5_hbm.at[jr, pl.ds(NB * i2, NB), jj], idxv.at[slot])
        for b in range(NB):
            for k in range(8):
                v = idxv[slot, b, pl.ds(16 * k, 16)]
                hbuf[slot, b, pl.ds(16 * k, 16)] = lax.shift_right_logical(v, 1)
                obuf[slot, b, pl.ds(16 * k, 16)] = lax.shift_left(v & 1, 6)
        for b in range(NB):
            pltpu.async_copy(w2_hbm.at[hbuf.at[slot, b]],
                             widebuf.at[slot, b], gsems[slot])

    def stage_b(t, slot):
        jr, jj, j, i2 = coords(t)
        for b in range(NB):
            pltpu.make_async_copy(w2_hbm.at[hbuf.at[slot, b]],
                                  widebuf.at[slot, b], gsems[slot]).wait()

        @pl.when(t >= 2)
        def _():
            for d in range(8):
                pltpu.make_async_copy(outbuf.at[slot, d],
                                      out5_hbm.at[j, d, pl.ds(NB * i2, NB)],
                                      ssems[slot]).wait()

        wide = widebuf.at[slot]

        def dblock(dmaj, carry):
            for b in range(NB):
                for k in range(8):
                    pvec = iota16 + 16 * k
                    ovec = obuf[slot, b, pl.ds(16 * k, 16)]
                    bvec = jnp.full((16,), b, jnp.int32)
                    for dd in range(8):
                        vals = plsc.load_gather(
                            wide, [bvec, pvec, ovec + (dmaj * 8 + dd)])
                        outbuf[slot, dmaj, b, dd, pl.ds(16 * k, 16)] = vals
            return carry

        lax.fori_loop(0, 8, dblock, 0)

        for d in range(8):
            pltpu.async_copy(outbuf.at[slot, d],
                             out5_hbm.at[j, d, pl.ds(NB * i2, NB)],
                             ssems[slot])

    stage_a(0, 0)

    def body(tt, carry):
        t0 = 2 * tt
        stage_a(t0 + 1, 1)
        stage_b(t0, 0)

        @pl.when(tt < SB_PER_W // 2 - 1)
        def _():
            stage_a(t0 + 2, 0)

        stage_b(t0 + 1, 1)
        return carry

    lax.fori_loop(0, SB_PER_W // 2, body, 0)

    _, _, j, i2 = coords(SB_PER_W - 2)
    for d in range(8):
        pltpu.make_async_copy(outbuf.at[0, d],
                              out5_hbm.at[j, d, pl.ds(NB * i2, NB)],
                              ssems[0]).wait()
    _, _, j, i2 = coords(SB_PER_W - 1)
    for d in range(8):
        pltpu.make_async_copy(outbuf.at[1, d],
                              out5_hbm.at[j, d, pl.ds(NB * i2, NB)],
                              ssems[1]).wait()


@jax.jit
def _embedding_lookup(x5, w2):
    mesh = plsc.VectorSubcoreMesh(core_axis_name="c", subcore_axis_name="s")
    k = functools.partial(
        pl.kernel,
        mesh=mesh,
        out_type=jax.ShapeDtypeStruct((200, 8, 32, 8, 128), jnp.float32),
        scratch_types=[
            pltpu.VMEM((2, NB, 128), jnp.int32),
            pltpu.VMEM((2, NB, 128), jnp.int32),
            pltpu.VMEM((2, NB, 128), jnp.int32),
            pltpu.VMEM((2, NB, 128, 128), jnp.float32),
            pltpu.VMEM((2, 8, NB, 8, 128), jnp.float32),
            pltpu.SemaphoreType.DMA,
            pltpu.SemaphoreType.DMA,
            pltpu.SemaphoreType.DMA,
            pltpu.SemaphoreType.DMA,
        ],
        compiler_params=pltpu.CompilerParams(needs_layout_passes=False),
    )(_emb_body)
    return k(x5, w2)


def kernel(x, weight):
    x5 = x.reshape(32, 128, 25, 8).transpose(2, 0, 3, 1)
    w2 = weight.reshape(500000, 128)
    out5 = _embedding_lookup(x5, w2)
    return out5.transpose(2, 4, 0, 1, 3).reshape(4096, 200, 64)


# speedup vs baseline: 1.8357x; 1.8357x over previous
import functools

import jax
import jax.numpy as jnp
from jax import lax
from jax.experimental import pallas as pl
from jax.experimental.pallas import tpu as pltpu
from jax.experimental.pallas import tpu_sc as plsc

NW = 32
NB = 2
SB_TOTAL = 200 * 16
SB_PER_W = SB_TOTAL // NW


def _emb_body(x5_hbm, w_hbm, out5_hbm,
              idxv, widebuf, outbuf,
              gsem0, gsem1, ssem0, ssem1):
    wid = lax.axis_index("s") * 2 + lax.axis_index("c")
    iota16 = lax.iota(jnp.int32, 16)
    gsems = (gsem0, gsem1)
    ssems = (ssem0, ssem1)

    def coords(t):
        s = wid * SB_PER_W + t
        j = s // 16
        i2 = s % 16
        return j // 8, j % 8, j, i2

    def stage_a(t, slot):
        jr, jj, j, i2 = coords(t)
        pltpu.sync_copy(x5_hbm.at[jr, pl.ds(NB * i2, NB), jj], idxv.at[slot])
        for b in range(NB):
            pltpu.async_copy(w_hbm.at[idxv.at[slot, b]],
                             widebuf.at[slot, b], gsems[slot])

    def stage_b(t, slot):
        jr, jj, j, i2 = coords(t)
        for b in range(NB):
            pltpu.make_async_copy(w_hbm.at[idxv.at[slot, b]],
                                  widebuf.at[slot, b], gsems[slot]).wait()

        @pl.when(t >= 2)
        def _():
            for b in range(NB):
                pltpu.make_async_copy(outbuf.at[slot, b],
                                      out5_hbm.at[j, :, NB * i2 + b],
                                      ssems[slot]).wait()

        wide = widebuf.at[slot]

        def diag(d0, carry):
            cvec = (iota16 + d0) & 63
            dmaj = lax.shift_right_logical(cvec, 3)
            dmin = cvec & 7
            for b in range(NB):
                bvec = jnp.full((16,), b, jnp.int32)
                for g in range(8):
                    rvec = iota16 + 16 * g
                    vals = plsc.load_gather(wide, [bvec, rvec, cvec])
                    plsc.store_scatter(outbuf.at[slot, b],
                                       [dmaj, dmin, rvec], vals)
            return carry

        lax.fori_loop(0, 64, diag, 0)

        for b in range(NB):
            pltpu.async_copy(outbuf.at[slot, b],
                             out5_hbm.at[j, :, NB * i2 + b],
                             ssems[slot])

    stage_a(0, 0)

    def body(tt, carry):
        t0 = 2 * tt
        stage_a(t0 + 1, 1)
        stage_b(t0, 0)

        @pl.when(tt < SB_PER_W // 2 - 1)
        def _():
            stage_a(t0 + 2, 0)

        stage_b(t0 + 1, 1)
        return carry

    lax.fori_loop(0, SB_PER_W // 2, body, 0)

    for slot in range(2):
        _, _, j, i2 = coords(SB_PER_W - 2 + slot)
        for b in range(NB):
            pltpu.make_async_copy(outbuf.at[slot, b],
                                  out5_hbm.at[j, :, NB * i2 + b],
                                  ssems[slot]).wait()


@jax.jit
def _embedding_lookup(x5, weight):
    mesh = plsc.VectorSubcoreMesh(core_axis_name="c", subcore_axis_name="s")
    k = functools.partial(
        pl.kernel,
        mesh=mesh,
        out_type=jax.ShapeDtypeStruct((200, 8, 32, 8, 128), jnp.float32),
        scratch_types=[
            pltpu.VMEM((2, NB, 128), jnp.int32),
            pltpu.VMEM((2, NB, 128, 64), jnp.float32),
            pltpu.VMEM((2, NB, 8, 8, 128), jnp.float32),
            pltpu.SemaphoreType.DMA,
            pltpu.SemaphoreType.DMA,
            pltpu.SemaphoreType.DMA,
            pltpu.SemaphoreType.DMA,
        ],
        compiler_params=pltpu.CompilerParams(
            use_tc_tiling_on_sc=False, needs_layout_passes=False),
    )(_emb_body)
    return k(x5, weight)


def kernel(x, weight):
    x5 = x.reshape(32, 128, 25, 8).transpose(2, 0, 3, 1)
    out5 = _embedding_lookup(x5, weight)
    return out5.transpose(2, 4, 0, 1, 3).reshape(4096, 200, 64)
